# baseline (speedup 1.0000x reference)
import functools

import jax
import jax.numpy as jnp
from jax import lax
from jax.experimental import pallas as pl
from jax.experimental.pallas import tpu as pltpu
from jax.experimental.pallas import tpu_sc as plsc

B = 16384
D = 64
H1, H2 = 200, 80

NC, NS = 2, 16
NW = NC * NS
B_PER_W = B // NW
CHUNK = 128
NCHUNK = B_PER_W // CHUNK


def _sc_gather(pos0, pos1, usr, user_table, item_table, cate_table):
  mesh = plsc.VectorSubcoreMesh(core_axis_name="c", subcore_axis_name="s")

  @functools.partial(
      pl.kernel,
      out_type=[
          jax.ShapeDtypeStruct((B, D), jnp.float32),
          jax.ShapeDtypeStruct((B, D), jnp.float32),
      ],
      mesh=mesh,
      compiler_params=pltpu.CompilerParams(use_tc_tiling_on_sc=False),
      scratch_types=[
          pltpu.VMEM((NCHUNK, CHUNK), jnp.int32),
          pltpu.VMEM((NCHUNK, CHUNK), jnp.int32),
          pltpu.VMEM((NCHUNK, CHUNK), jnp.int32),
          pltpu.VMEM((B_PER_W, D), jnp.float32),
          pltpu.VMEM((B_PER_W, D), jnp.float32),
          pltpu.SemaphoreType.DMA,
          pltpu.SemaphoreType.DMA,
      ],
  )
  def gather_kernel(pos0_h, pos1_h, usr_h, ut_h, it_h, ct_h,
                    out_pos_h, out_usr_h,
                    idx_i, idx_c, idx_u, rows_p, rows_u, sem, sem2):
    wid = lax.axis_index("s") * NC + lax.axis_index("c")
    r0 = wid * NCHUNK
    pltpu.sync_copy(pos0_h.at[pl.ds(r0, NCHUNK)], idx_i)
    pltpu.sync_copy(pos1_h.at[pl.ds(r0, NCHUNK)], idx_c)
    pltpu.sync_copy(usr_h.at[pl.ds(r0, NCHUNK)], idx_u)
    cps = []
    for j in range(NCHUNK):
      dst = rows_p.at[pl.ds(j * CHUNK, CHUNK)]
      cps.append(pltpu.async_copy(it_h.at[idx_i.at[j]], dst, sem))
      dstu = rows_u.at[pl.ds(j * CHUNK, CHUNK)]
      cps.append(pltpu.async_copy(ut_h.at[idx_u.at[j]], dstu, sem))
    for cp in cps:
      cp.wait()
    cps2 = []
    for j in range(NCHUNK):
      dst = rows_p.at[pl.ds(j * CHUNK, CHUNK)]
      cps2.append(pltpu.async_copy(ct_h.at[idx_c.at[j]], dst, sem2, add=True))
    for cp in cps2:
      cp.wait()
    base = wid * B_PER_W
    pltpu.sync_copy(rows_p, out_pos_h.at[pl.ds(base, B_PER_W)])
    pltpu.sync_copy(rows_u, out_usr_h.at[pl.ds(base, B_PER_W)])

  return gather_kernel(pos0, pos1, usr, user_table, item_table, cate_table)


BK = 2048


def _mlp_body(pos_ref, usr_ref, w1a_ref, w1b_ref, b1_ref, w2_ref, b2_ref,
              w3_ref, b3_ref, out_ref):
  h = jnp.dot(pos_ref[...], w1a_ref[...], preferred_element_type=jnp.float32)
  h = h + jnp.dot(usr_ref[...], w1b_ref[...],
                  preferred_element_type=jnp.float32)
  h = jnp.maximum(h + b1_ref[...], 0.0)
  h = jnp.maximum(jnp.dot(h, w2_ref[...], preferred_element_type=jnp.float32)
                  + b2_ref[...], 0.0)
  logit = jnp.sum(h * w3_ref[...], axis=1, keepdims=True) + b3_ref[...]
  out_ref[...] = jax.nn.sigmoid(logit)


def _tc_mlp(pos_emb, user_emb, W1, b1, W2, b2, W3, b3):
  w1a, w1b = W1[:D], W1[D:]
  b1r = b1.reshape(1, H1)
  b2r = b2.reshape(1, H2)
  w3r = W3.reshape(1, H2)
  b3r = b3.reshape(1, 1)
  full = lambda shape: pl.BlockSpec(shape, lambda i: (0,) * len(shape))
  out = pl.pallas_call(
      _mlp_body,
      grid=(B // BK,),
      in_specs=[
          pl.BlockSpec((BK, D), lambda i: (i, 0)),
          pl.BlockSpec((BK, D), lambda i: (i, 0)),
          full((D, H1)),
          full((D, H1)),
          full((1, H1)),
          full((H1, H2)),
          full((1, H2)),
          full((1, H2)),
          full((1, 1)),
      ],
      out_specs=pl.BlockSpec((BK, 1), lambda i: (i, 0)),
      out_shape=jax.ShapeDtypeStruct((B, 1), jnp.float32),
  )(pos_emb, user_emb, w1a, w1b, b1r, W2, b2r, w3r, b3r)
  return out[:, 0]


def kernel(user, rec_his, satis_his, dissatis_his, pos_item, neg_items,
           user_table, item_table, cate_table, W1, b1, W2, b2, W3, b3):
  pos0 = pos_item[0].reshape(B // CHUNK, CHUNK)
  pos1 = pos_item[1].reshape(B // CHUNK, CHUNK)
  usr = user.reshape(B // CHUNK, CHUNK)
  pos_emb, user_emb = _sc_gather(pos0, pos1, usr,
                                 user_table, item_table, cate_table)
  return _tc_mlp(pos_emb, user_emb, W1, b1, W2, b2, W3, b3)


# speedup vs baseline: 1.5158x; 1.5158x over previous
import functools

import jax
import jax.numpy as jnp
from jax import lax
from jax.experimental import pallas as pl
from jax.experimental.pallas import tpu as pltpu
from jax.experimental.pallas import tpu_sc as plsc

B = 16384
D = 64
H1, H2 = 200, 80

NC, NS = 2, 16
NW = NC * NS
B_PER_W = B // NW
CHUNK = 64
NCHUNK = B_PER_W // CHUNK


def _sc_gather(pos0, pos1, usr, user_table, item_table, cate_table):
  mesh = plsc.VectorSubcoreMesh(core_axis_name="c", subcore_axis_name="s")

  @functools.partial(
      pl.kernel,
      out_type=[
          jax.ShapeDtypeStruct((B, D), jnp.float32),
          jax.ShapeDtypeStruct((B, D), jnp.float32),
      ],
      mesh=mesh,
      compiler_params=pltpu.CompilerParams(use_tc_tiling_on_sc=True),
      scratch_types=[
          pltpu.VMEM((B_PER_W,), jnp.int32),
          pltpu.VMEM((B_PER_W,), jnp.int32),
          pltpu.VMEM((B_PER_W,), jnp.int32),
          pltpu.VMEM((CHUNK, D), jnp.float32),
          pltpu.VMEM((CHUNK, D), jnp.float32),
          pltpu.VMEM((CHUNK, D), jnp.float32),
          pltpu.SemaphoreType.DMA,
          pltpu.SemaphoreType.DMA,
      ],
  )
  def gather_kernel(pos0_h, pos1_h, usr_h, ut_h, it_h, ct_h,
                    out_pos_h, out_usr_h,
                    idx_i, idx_c, idx_u, buf_a, buf_b, buf_o, sem, sem2):
    wid = lax.axis_index("s") * NC + lax.axis_index("c")
    base = wid * B_PER_W
    pltpu.sync_copy(pos0_h.at[pl.ds(base, B_PER_W)], idx_i)
    pltpu.sync_copy(pos1_h.at[pl.ds(base, B_PER_W)], idx_c)
    pltpu.sync_copy(usr_h.at[pl.ds(base, B_PER_W)], idx_u)

    def item_chunk(c, _):
      def fire(j, _):
        vi = idx_i[pl.ds(c * CHUNK + j * 16, 16)]
        vc = idx_c[pl.ds(c * CHUNK + j * 16, 16)]
        for r in range(16):
          pltpu.async_copy(it_h.at[pl.ds(vi[r], 1)],
                           buf_a.at[pl.ds(j * 16 + r, 1)], sem)
          pltpu.async_copy(ct_h.at[pl.ds(vc[r], 1)],
                           buf_b.at[pl.ds(j * 16 + r, 1)], sem2)
        return 0

      lax.fori_loop(0, CHUNK // 16, fire, 0)

      def drain(r, _):
        pltpu.make_async_copy(it_h.at[pl.ds(0, 1)],
                              buf_a.at[pl.ds(0, 1)], sem).wait()
        pltpu.make_async_copy(ct_h.at[pl.ds(0, 1)],
                              buf_b.at[pl.ds(0, 1)], sem2).wait()
        return 0

      lax.fori_loop(0, CHUNK, drain, 0)

      def add_row(r, _):
        for k in range(D // 16):
          sl = pl.ds(k * 16, 16)
          buf_o[r, sl] = buf_a[r, sl] + buf_b[r, sl]
        return 0

      lax.fori_loop(0, CHUNK, add_row, 0)
      pltpu.sync_copy(buf_o, out_pos_h.at[pl.ds(base + c * CHUNK, CHUNK)])
      return 0

    lax.fori_loop(0, NCHUNK, item_chunk, 0)

    def user_chunk(c, _):
      def fire(j, _):
        vu = idx_u[pl.ds(c * CHUNK + j * 16, 16)]
        for r in range(16):
          pltpu.async_copy(ut_h.at[pl.ds(vu[r], 1)],
                           buf_a.at[pl.ds(j * 16 + r, 1)], sem)
        return 0

      lax.fori_loop(0, CHUNK // 16, fire, 0)

      def drain(r, _):
        pltpu.make_async_copy(ut_h.at[pl.ds(0, 1)],
                              buf_a.at[pl.ds(0, 1)], sem).wait()
        return 0

      lax.fori_loop(0, CHUNK, drain, 0)
      pltpu.sync_copy(buf_a, out_usr_h.at[pl.ds(base + c * CHUNK, CHUNK)])
      return 0

    lax.fori_loop(0, NCHUNK, user_chunk, 0)

  return gather_kernel(pos0, pos1, usr, user_table, item_table, cate_table)


BK = 2048


def _mlp_body(pos_ref, usr_ref, w1a_ref, w1b_ref, b1_ref, w2_ref, b2_ref,
              w3_ref, b3_ref, out_ref):
  h = jnp.dot(pos_ref[...], w1a_ref[...], preferred_element_type=jnp.float32)
  h = h + jnp.dot(usr_ref[...], w1b_ref[...],
                  preferred_element_type=jnp.float32)
  h = jnp.maximum(h + b1_ref[...], 0.0)
  h = jnp.maximum(jnp.dot(h, w2_ref[...], preferred_element_type=jnp.float32)
                  + b2_ref[...], 0.0)
  logit = jnp.sum(h * w3_ref[...], axis=1, keepdims=True) + b3_ref[...]
  out_ref[...] = jax.nn.sigmoid(logit)


def _tc_mlp(pos_emb, user_emb, W1, b1, W2, b2, W3, b3):
  w1a, w1b = W1[:D], W1[D:]
  b1r = b1.reshape(1, H1)
  b2r = b2.reshape(1, H2)
  w3r = W3.reshape(1, H2)
  b3r = b3.reshape(1, 1)
  full = lambda shape: pl.BlockSpec(shape, lambda i: (0,) * len(shape))
  out = pl.pallas_call(
      _mlp_body,
      grid=(B // BK,),
      in_specs=[
          pl.BlockSpec((BK, D), lambda i: (i, 0)),
          pl.BlockSpec((BK, D), lambda i: (i, 0)),
          full((D, H1)),
          full((D, H1)),
          full((1, H1)),
          full((H1, H2)),
          full((1, H2)),
          full((1, H2)),
          full((1, 1)),
      ],
      out_specs=pl.BlockSpec((BK, 1), lambda i: (i, 0)),
      out_shape=jax.ShapeDtypeStruct((B, 1), jnp.float32),
  )(pos_emb, user_emb, w1a, w1b, b1r, W2, b2r, w3r, b3r)
  return out[:, 0]


def kernel(user, rec_his, satis_his, dissatis_his, pos_item, neg_items,
           user_table, item_table, cate_table, W1, b1, W2, b2, W3, b3):
  pos_emb, user_emb = _sc_gather(pos_item[0], pos_item[1], user,
                                 user_table, item_table, cate_table)
  return _tc_mlp(pos_emb, user_emb, W1, b1, W2, b2, W3, b3)
